# Initial kernel scaffold; baseline (speedup 1.0000x reference)
import jax
import jax.numpy as jnp
from jax.experimental import pallas as pl


def kernel(features, edge_index, bn_gamma, bn_beta, W_fc, W_l, W_r):
    raise NotImplementedError("write your pallas kernel here")


# speedup vs baseline: 28.8248x; 28.8248x over previous
import functools

import jax
import jax.numpy as jnp
from jax import lax
from jax.experimental import pallas as pl
from jax.experimental.pallas import tpu as pltpu
from jax.experimental.pallas import tpu_sc as plsc

N = 10000
E = 320000
IN_CH = 128
OUT = 64
NH = 2
FT = 128

CHUNK = 128
CPS = 157
PAD_E = 16 * CPS * CHUNK
NACC = 10240
RPS = NACC // 16


def _prep_body(x_ref, g_ref, b_ref, wfc_ref, wl_ref, wr_ref,
               t_ref, a1_ref, a2_ref):
    x = x_ref[...]
    mean = jnp.mean(x, axis=0, keepdims=True)
    xc = x - mean
    var = jnp.mean(xc * xc, axis=0, keepdims=True)
    y = xc * (g_ref[...] * lax.rsqrt(var + 1e-5)) + b_ref[...]
    wfc0 = wfc_ref[0]
    wfc1 = wfc_ref[1]
    ftc = jnp.dot(y, jnp.concatenate([wfc0, wfc1], axis=1),
                  preferred_element_type=jnp.float32)
    ones = jnp.ones((N, 1), jnp.float32)
    zpad = jnp.zeros((N, FT - OUT - 1), jnp.float32)
    zrows = jnp.zeros((NACC - N, FT), jnp.float32)
    t_ref[...] = jnp.concatenate([
        jnp.concatenate([ftc[:, :OUT], ones, zpad], axis=1), zrows,
        jnp.concatenate([ftc[:, OUT:], ones, zpad], axis=1), zrows,
    ], axis=0)
    uv = jnp.concatenate(
        [jnp.dot(wfc0, wl_ref[0]), jnp.dot(wfc0, wr_ref[0]),
         jnp.dot(wfc1, wl_ref[1]), jnp.dot(wfc1, wr_ref[1])], axis=1)
    a = jnp.dot(y, uv, preferred_element_type=jnp.float32)
    pad = jnp.zeros((2, NACC - N), jnp.float32)
    a1_ref[...] = jnp.concatenate([jnp.stack([a[:, 0], a[:, 2]]), pad], axis=1)
    a2_ref[...] = jnp.concatenate([jnp.stack([a[:, 1], a[:, 3]]), pad], axis=1)


_prep = pl.pallas_call(
    _prep_body,
    out_shape=[
        jax.ShapeDtypeStruct((2 * NACC, FT), jnp.float32),
        jax.ShapeDtypeStruct((2, NACC), jnp.float32),
        jax.ShapeDtypeStruct((2, NACC), jnp.float32),
    ],
)


_mesh = plsc.VectorSubcoreMesh(core_axis_name="c", subcore_axis_name="s")


@functools.partial(
    pl.kernel,
    out_type=jax.ShapeDtypeStruct((2, NACC, FT), jnp.float32),
    mesh=_mesh,
    compiler_params=pltpu.CompilerParams(needs_layout_passes=False),
    scratch_types=[
        pltpu.VMEM((NACC,), jnp.float32),
        pltpu.VMEM((NACC,), jnp.float32),
        pltpu.VMEM((CHUNK,), jnp.int32),
        pltpu.VMEM((CHUNK,), jnp.int32),
        pltpu.VMEM((CHUNK,), jnp.float32),
        pltpu.VMEM((CHUNK, FT), jnp.float32),
        pltpu.VMEM_SHARED((NACC, FT), jnp.float32),
    ],
)
def _edge_pass(src_hbm, dst_hbm, t_hbm, a1_hbm, a2_hbm, num_out,
               a1v, a2v, srcv, dstv, sb, ftb, num_sh):
    cid = lax.axis_index("c")
    sid = lax.axis_index("s")

    pltpu.sync_copy(a1_hbm.at[cid], a1v)
    pltpu.sync_copy(a2_hbm.at[cid], a2v)

    zero16 = jnp.zeros((16,), jnp.float32)

    def _zrow(j, _):
        for q in range(FT // 16):
            ftb[j, pl.ds(q * 16, 16)] = zero16
        return 0

    lax.fori_loop(0, CHUNK, _zrow, 0)

    row0 = sid * RPS
    for b in range(RPS // CHUNK):
        pltpu.sync_copy(ftb, num_sh.at[pl.ds(row0 + b * CHUNK, CHUNK)])
    plsc.subcore_barrier()

    tbase = cid * NACC

    def _chunk(k, _):
        base = (sid * CPS + k) * CHUNK
        pltpu.sync_copy(src_hbm.at[pl.ds(base, CHUNK)], srcv)
        pltpu.sync_copy(dst_hbm.at[pl.ds(base, CHUNK)], dstv)

        def _weights(g, _):
            sl = pl.ds(g * 16, 16)
            s16 = srcv[sl]
            z = plsc.load_gather(a1v, [dstv[sl]]) + plsc.load_gather(a2v, [s16])
            z = jnp.where(z >= 0.0, z, 0.01 * z)
            sb[sl] = jnp.exp(z)
            srcv[sl] = s16 + tbase
            return 0

        lax.fori_loop(0, CHUNK // 16, _weights, 0)
        pltpu.sync_copy(t_hbm.at[srcv], ftb)

        def _scale(g, _):
            sv = sb[pl.ds(g * 16, 16)]
            for j in range(16):
                row = g * 16 + j
                s = sv[j]
                for q in range(OUT // 16 + 1):
                    qsl = pl.ds(q * 16, 16)
                    ftb[row, qsl] = ftb[row, qsl] * s
            return 0

        lax.fori_loop(0, CHUNK // 16, _scale, 0)
        pltpu.sync_copy(ftb, num_sh.at[dstv], add=True)
        return 0

    lax.fori_loop(0, CPS, _chunk, 0)
    plsc.subcore_barrier()

    rows = pl.ds(row0, RPS)
    pltpu.sync_copy(num_sh.at[rows], num_out.at[cid, rows])


def _fin_body(num_ref, out_ref):
    for h in range(NH):
        num = num_ref[h, :N, :OUT]
        den = num_ref[h, :N, OUT]
        out_ref[:, h * OUT:(h + 1) * OUT] = jnp.maximum(
            num / jnp.maximum(den, 1e-16)[:, None], 0.0)


_finalize = pl.pallas_call(
    _fin_body,
    out_shape=jax.ShapeDtypeStruct((N, NH * OUT), jnp.float32),
)


def kernel(features, edge_index, bn_gamma, bn_beta, W_fc, W_l, W_r):
    t, a1, a2 = _prep(features, bn_gamma.reshape(1, IN_CH),
                      bn_beta.reshape(1, IN_CH), W_fc, W_l, W_r)
    npad = PAD_E - E
    src = jnp.concatenate([edge_index[0], jnp.zeros((npad,), jnp.int32)])
    dst = jnp.concatenate(
        [edge_index[1], N + (jnp.arange(npad, dtype=jnp.int32) % (NACC - N))])
    num = _edge_pass(src, dst, t, a1, a2)
    return _finalize(num)


# speedup vs baseline: 36.6971x; 1.2731x over previous
import functools

import jax
import jax.numpy as jnp
from jax import lax
from jax.experimental import pallas as pl
from jax.experimental.pallas import tpu as pltpu
from jax.experimental.pallas import tpu_sc as plsc

N = 10000
E = 320000
IN_CH = 128
OUT = 64
NH = 2
FT = 128

CHUNK = 64
CPS = 315
PAD_E = 16 * CPS * CHUNK
NACC = 10112
RPS = NACC // 16
BUFS = 3


def _prep_body(x_ref, g_ref, b_ref, wfc_ref, wl_ref, wr_ref,
               t_ref, a1_ref, a2_ref):
    x = x_ref[...]
    mean = jnp.mean(x, axis=0, keepdims=True)
    xc = x - mean
    var = jnp.mean(xc * xc, axis=0, keepdims=True)
    y = xc * (g_ref[...] * lax.rsqrt(var + 1e-5)) + b_ref[...]
    wfc0 = wfc_ref[0]
    wfc1 = wfc_ref[1]
    ftc = jnp.dot(y, jnp.concatenate([wfc0, wfc1], axis=1),
                  preferred_element_type=jnp.float32)
    ones = jnp.ones((N, 1), jnp.float32)
    zpad = jnp.zeros((N, FT - OUT - 1), jnp.float32)
    zrows = jnp.zeros((NACC - N, FT), jnp.float32)
    t_ref[...] = jnp.concatenate([
        jnp.concatenate([ftc[:, :OUT], ones, zpad], axis=1), zrows,
        jnp.concatenate([ftc[:, OUT:], ones, zpad], axis=1), zrows,
    ], axis=0)
    uv = jnp.concatenate(
        [jnp.dot(wfc0, wl_ref[0]), jnp.dot(wfc0, wr_ref[0]),
         jnp.dot(wfc1, wl_ref[1]), jnp.dot(wfc1, wr_ref[1])], axis=1)
    a = jnp.dot(y, uv, preferred_element_type=jnp.float32)
    pad = jnp.zeros((2, NACC - N), jnp.float32)
    a1_ref[...] = jnp.concatenate([jnp.stack([a[:, 0], a[:, 2]]), pad], axis=1)
    a2_ref[...] = jnp.concatenate([jnp.stack([a[:, 1], a[:, 3]]), pad], axis=1)


_prep = pl.pallas_call(
    _prep_body,
    out_shape=[
        jax.ShapeDtypeStruct((2 * NACC, FT), jnp.float32),
        jax.ShapeDtypeStruct((2, NACC), jnp.float32),
        jax.ShapeDtypeStruct((2, NACC), jnp.float32),
    ],
)


_mesh = plsc.VectorSubcoreMesh(core_axis_name="c", subcore_axis_name="s")


@functools.partial(
    pl.kernel,
    out_type=jax.ShapeDtypeStruct((2, NACC, FT), jnp.float32),
    mesh=_mesh,
    compiler_params=pltpu.CompilerParams(needs_layout_passes=False),
    scratch_types=[
        pltpu.VMEM((NACC,), jnp.float32),
        pltpu.VMEM((NACC,), jnp.float32),
        pltpu.VMEM((BUFS, 2, CHUNK), jnp.int32),
        pltpu.VMEM((BUFS, CHUNK), jnp.float32),
        pltpu.VMEM((BUFS, CHUNK, FT), jnp.float32),
        pltpu.VMEM_SHARED((NACC, FT), jnp.float32),
        pltpu.SemaphoreType.DMA,
        pltpu.SemaphoreType.DMA,
        pltpu.SemaphoreType.DMA,
        pltpu.SemaphoreType.DMA,
        pltpu.SemaphoreType.DMA,
        pltpu.SemaphoreType.DMA,
    ],
)
def _edge_pass(idx_hbm, t_hbm, a1_hbm, a2_hbm, num_out,
               a1v, a2v, idxb, sb, ftb, num_sh,
               gs0, gs1, gs2, ss0, ss1, ss2):
    gsem = (gs0, gs1, gs2)
    ssem = (ss0, ss1, ss2)
    cid = lax.axis_index("c")
    sid = lax.axis_index("s")

    pltpu.sync_copy(a1_hbm.at[cid], a1v)
    pltpu.sync_copy(a2_hbm.at[cid], a2v)

    zero16 = jnp.zeros((16,), jnp.float32)

    def _zrow(j, _):
        for q in range(FT // 16):
            ftb[0, j, pl.ds(q * 16, 16)] = zero16
        return 0

    lax.fori_loop(0, CHUNK, _zrow, 0)

    row0 = sid * RPS
    for b in range(RPS // CHUNK):
        pltpu.sync_copy(ftb.at[0],
                        num_sh.at[pl.ds(row0 + b * CHUNK, CHUNK)])
    rem = RPS - (RPS // CHUNK) * CHUNK
    if rem:
        pltpu.sync_copy(ftb.at[0].at[pl.ds(0, rem)],
                        num_sh.at[pl.ds(row0 + RPS - rem, rem)])
    plsc.subcore_barrier()

    tbase = cid * NACC
    base0 = sid * CPS

    def _prefetch(kg, p):
        pltpu.sync_copy(idx_hbm.at[kg], idxb.at[p])

        def _w(g, _):
            sl = pl.ds(g * 16, 16)
            s16 = idxb[p, 0, sl]
            z = (plsc.load_gather(a1v, [idxb[p, 1, sl]])
                 + plsc.load_gather(a2v, [s16]))
            z = jnp.where(z >= 0.0, z, 0.01 * z)
            sb[p, sl] = jnp.exp(z)
            idxb[p, 0, sl] = s16 + tbase
            return 0

        lax.fori_loop(0, CHUNK // 16, _w, 0)
        pltpu.async_copy(t_hbm.at[idxb.at[p, 0]], ftb.at[p], gsem[p])

    def _wait_gather(p):
        pltpu.make_async_copy(t_hbm.at[idxb.at[p, 0]], ftb.at[p],
                              gsem[p]).wait()

    def _scale(p):
        def _s(g, _):
            sv = sb[p, pl.ds(g * 16, 16)]
            for j in range(16):
                row = g * 16 + j
                s = sv[j]
                for q in range(OUT // 16 + 1):
                    qsl = pl.ds(q * 16, 16)
                    ftb[p, row, qsl] = ftb[p, row, qsl] * s
            return 0

        lax.fori_loop(0, CHUNK // 16, _s, 0)

    def _issue_scatter(p):
        pltpu.async_copy(ftb.at[p], num_sh.at[idxb.at[p, 1]], ssem[p],
                         add=True)

    def _wait_scatter(p):
        pltpu.make_async_copy(ftb.at[p], num_sh.at[idxb.at[p, 1]],
                              ssem[p]).wait()

    _prefetch(base0, 0)
    _prefetch(base0 + 1, 1)

    def _iter(i, _):
        for p in range(BUFS):
            j = i * BUFS + p
            pn = (p + 2) % BUFS
            @pl.when(jnp.logical_and(j >= 1, j + 2 < CPS))
            def _():
                _wait_scatter(pn)

            @pl.when(j + 2 < CPS)
            def _():
                _prefetch(base0 + j + 2, pn)

            _wait_gather(p)
            _scale(p)
            _issue_scatter(p)
        return 0

    lax.fori_loop(0, CPS // BUFS, _iter, 0)
    for p in range(BUFS):
        _wait_scatter(p)
    plsc.subcore_barrier()

    rows = pl.ds(row0, RPS)
    pltpu.sync_copy(num_sh.at[rows], num_out.at[cid, rows])


def _fin_body(num_ref, out_ref):
    for h in range(NH):
        num = num_ref[h, :N, :OUT]
        den = num_ref[h, :N, OUT]
        out_ref[:, h * OUT:(h + 1) * OUT] = jnp.maximum(
            num / jnp.maximum(den, 1e-16)[:, None], 0.0)


_finalize = pl.pallas_call(
    _fin_body,
    out_shape=jax.ShapeDtypeStruct((N, NH * OUT), jnp.float32),
)


def kernel(features, edge_index, bn_gamma, bn_beta, W_fc, W_l, W_r):
    t, a1, a2 = _prep(features, bn_gamma.reshape(1, IN_CH),
                      bn_beta.reshape(1, IN_CH), W_fc, W_l, W_r)
    npad = PAD_E - E
    src = jnp.concatenate([edge_index[0], jnp.zeros((npad,), jnp.int32)])
    dst = jnp.concatenate(
        [edge_index[1], N + (jnp.arange(npad, dtype=jnp.int32) % (NACC - N))])
    nch = PAD_E // CHUNK
    idx = jnp.stack([src.reshape(nch, CHUNK), dst.reshape(nch, CHUNK)], axis=1)
    num = _edge_pass(idx, t, a1, a2)
    return _finalize(num)
